# baseline (speedup 1.0000x reference)
import functools
import math

import jax
import jax.numpy as jnp
from jax import lax
from jax.experimental import pallas as pl
from jax.experimental.pallas import tpu as pltpu
from jax.experimental.pallas import tpu_sc as plsc

_SCALE = 64.0
_INV_SCALE = 1.0 / 64.0
_MARGIN = 0.5
_COS_M = math.cos(_MARGIN)
_SIN_M = math.sin(_MARGIN)

_NC = 2
_NS = 16
_NW = _NC * _NS
_LANES = 16

_N = 1024
_C = 100000
_CR = 32
_NCHUNK = _C // _CR
_TSTEPS = 99
_NGRP = _N // _LANES


def _fix_from_x(x):
    a = jnp.maximum(1.0 - x * x, 1e-12)
    bits = lax.bitcast_convert_type(a, jnp.int32)
    y = lax.bitcast_convert_type((bits >> 1) + 0x1FBD1DF5, jnp.float32)
    for _ in range(3):
        y = 0.5 * (y + a / y)
    return _SCALE * (x * _COS_M - y * _SIN_M)


def _sc_arc_margin_t(xt, label):
    mesh = plsc.VectorSubcoreMesh(core_axis_name="c", subcore_axis_name="s")

    @functools.partial(
        pl.kernel,
        mesh=mesh,
        compiler_params=pltpu.CompilerParams(needs_layout_passes=False),
        out_type=jax.ShapeDtypeStruct((_C, _N), jnp.float32),
        scratch_types=[
            pltpu.VMEM((_N,), jnp.int32),
            pltpu.VMEM((_CR, _N), jnp.float32),
            pltpu.VMEM((_CR, _N), jnp.float32),
            pltpu.VMEM((_CR, _N), jnp.float32),
            pltpu.SemaphoreType.DMA,
            pltpu.SemaphoreType.DMA,
            pltpu.SemaphoreType.DMA,
            pltpu.SemaphoreType.DMA,
            pltpu.SemaphoreType.DMA,
            pltpu.SemaphoreType.DMA,
        ],
    )
    def k(x_hbm, lbl_hbm, out_hbm, lbl_v, buf0, buf1, buf2,
          si0, si1, si2, so0, so1, so2):
        wid = lax.axis_index("s") * _NC + lax.axis_index("c")

        pltpu.sync_copy(lbl_hbm, lbl_v)

        bufs = (buf0, buf1, buf2)
        sins = (si0, si1, si2)
        souts = (so0, so1, so2)

        def load(c, b):
            pltpu.async_copy(x_hbm.at[pl.ds(c * _CR, _CR), :], bufs[b], sins[b])

        def wait_load(c, b):
            pltpu.make_async_copy(x_hbm.at[pl.ds(c * _CR, _CR), :],
                                  bufs[b], sins[b]).wait()

        def store(c, b):
            pltpu.async_copy(bufs[b], out_hbm.at[pl.ds(c * _CR, _CR), :],
                             souts[b])

        def wait_store(c, b):
            pltpu.make_async_copy(bufs[b], out_hbm.at[pl.ds(c * _CR, _CR), :],
                                  souts[b]).wait()

        def process(c, b):
            buf = bufs[b]
            r0 = c * _CR

            @plsc.parallel_loop(0, _CR)
            def _row(r):
                for i in range(_N // _LANES):
                    buf[r, pl.ds(i * _LANES, _LANES)] = (
                        buf[r, pl.ds(i * _LANES, _LANES)] * _SCALE)

            @pl.loop(0, _NGRP)
            def _grp(jv):
                lbl = lbl_v[pl.ds(jv * _LANES, _LANES)]
                off = lbl - r0
                m = (off >= 0) & (off < _CR)
                hits = plsc.all_reduce_population_count(m)

                @pl.when(jnp.max(hits) > 0)
                def _():
                    colv = lax.iota(jnp.int32, _LANES) + jv * _LANES
                    offr = jnp.minimum(jnp.maximum(off, 0), _CR - 1)
                    y = plsc.load_gather(buf, [offr, colv], mask=m)
                    fx = _fix_from_x(y * _INV_SCALE)
                    plsc.store_scatter(buf, [offr, colv], fx, mask=m)

        start = 97 * wid + jnp.minimum(wid, 21)
        nw = jnp.where(wid < 21, 98, 97)

        load(start, 0)
        load(start + 1, 1)

        @pl.loop(0, _TSTEPS // 3)
        def _trip(g):
            for b3 in range(3):
                t = 3 * g + b3
                b = b3
                bp = (b3 + 2) % 3
                c = start + t

                @pl.when(t < nw)
                def _():
                    wait_load(c, b)
                    process(c, b)
                    store(c, b)

                @pl.when((t >= 1) & (t - 1 < nw))
                def _():
                    wait_store(c - 1, bp)

                @pl.when(t + 2 < nw)
                def _():
                    load(c + 2, bp)

    return k(xt, label)


def kernel(cosine, label):
    out_t = _sc_arc_margin_t(cosine.T, label.astype(jnp.int32))
    return out_t.T


# speedup vs baseline: 1.1645x; 1.1645x over previous
import functools
import math

import jax
import jax.numpy as jnp
from jax import lax
from jax.experimental import pallas as pl
from jax.experimental.pallas import tpu as pltpu
from jax.experimental.pallas import tpu_sc as plsc

_SCALE = 64.0
_INV_SCALE = 1.0 / 64.0
_MARGIN = 0.5
_COS_M = math.cos(_MARGIN)
_SIN_M = math.sin(_MARGIN)

_NC = 2
_NS = 16
_NW = _NC * _NS
_LANES = 16

_N = 1024
_C = 100000
_CR = 40
_NCHUNK = _C // _CR
_TSTEPS = 81
_NGRP = _N // _LANES


def _fix_from_x(x):
    a = jnp.maximum(1.0 - x * x, 1e-12)
    bits = lax.bitcast_convert_type(a, jnp.int32)
    y = lax.bitcast_convert_type((bits >> 1) + 0x1FBD1DF5, jnp.float32)
    for _ in range(3):
        y = 0.5 * (y + a / y)
    return _SCALE * (x * _COS_M - y * _SIN_M)


def _sc_arc_margin_t(xt, label):
    mesh = plsc.VectorSubcoreMesh(core_axis_name="c", subcore_axis_name="s")

    @functools.partial(
        pl.kernel,
        mesh=mesh,
        compiler_params=pltpu.CompilerParams(needs_layout_passes=False),
        out_type=jax.ShapeDtypeStruct((_C, _N), jnp.float32),
        scratch_types=[
            pltpu.VMEM((_N,), jnp.int32),
            pltpu.VMEM((_CR, _N), jnp.float32),
            pltpu.VMEM((_CR, _N), jnp.float32),
            pltpu.VMEM((_CR, _N), jnp.float32),
            pltpu.SemaphoreType.DMA,
            pltpu.SemaphoreType.DMA,
            pltpu.SemaphoreType.DMA,
            pltpu.SemaphoreType.DMA,
            pltpu.SemaphoreType.DMA,
            pltpu.SemaphoreType.DMA,
        ],
    )
    def k(x_hbm, lbl_hbm, out_hbm, lbl_v, buf0, buf1, buf2,
          si0, si1, si2, so0, so1, so2):
        wid = lax.axis_index("s") * _NC + lax.axis_index("c")

        pltpu.sync_copy(lbl_hbm, lbl_v)

        bufs = (buf0, buf1, buf2)
        sins = (si0, si1, si2)
        souts = (so0, so1, so2)

        def load(c, b):
            pltpu.async_copy(x_hbm.at[pl.ds(c * _CR, _CR), :], bufs[b], sins[b])

        def wait_load(c, b):
            pltpu.make_async_copy(x_hbm.at[pl.ds(c * _CR, _CR), :],
                                  bufs[b], sins[b]).wait()

        def store(c, b):
            pltpu.async_copy(bufs[b], out_hbm.at[pl.ds(c * _CR, _CR), :],
                             souts[b])

        def wait_store(c, b):
            pltpu.make_async_copy(bufs[b], out_hbm.at[pl.ds(c * _CR, _CR), :],
                                  souts[b]).wait()

        def process(c, b):
            buf = bufs[b]
            r0 = c * _CR

            @plsc.parallel_loop(0, _CR)
            def _row(r):
                for i in range(_N // _LANES):
                    buf[r, pl.ds(i * _LANES, _LANES)] = (
                        buf[r, pl.ds(i * _LANES, _LANES)] * _SCALE)

            @pl.loop(0, _NGRP)
            def _grp(jv):
                lbl = lbl_v[pl.ds(jv * _LANES, _LANES)]
                off = lbl - r0
                m = (off >= 0) & (off < _CR)
                hits = plsc.all_reduce_population_count(m)

                @pl.when(jnp.max(hits) > 0)
                def _():
                    colv = lax.iota(jnp.int32, _LANES) + jv * _LANES
                    offr = jnp.minimum(jnp.maximum(off, 0), _CR - 1)
                    y = plsc.load_gather(buf, [offr, colv], mask=m)
                    fx = _fix_from_x(y * _INV_SCALE)
                    plsc.store_scatter(buf, [offr, colv], fx, mask=m)

        start = 78 * wid + jnp.minimum(wid, 4)
        nw = jnp.where(wid < 4, 79, 78)

        load(start, 0)
        load(start + 1, 1)

        @pl.loop(0, _TSTEPS // 3)
        def _trip(g):
            for b3 in range(3):
                t = 3 * g + b3
                b = b3
                bp = (b3 + 2) % 3
                c = start + t

                @pl.when(t < nw)
                def _():
                    wait_load(c, b)
                    process(c, b)
                    store(c, b)

                @pl.when((t >= 1) & (t - 1 < nw))
                def _():
                    wait_store(c - 1, bp)

                @pl.when(t + 2 < nw)
                def _():
                    load(c + 2, bp)

    return k(xt, label)


def kernel(cosine, label):
    out_t = _sc_arc_margin_t(cosine.T, label.astype(jnp.int32))
    return out_t.T
